# baseline (speedup 1.0000x reference)
import functools

import jax
import jax.numpy as jnp
from jax import lax
from jax.experimental import pallas as pl
from jax.experimental.pallas import tpu as pltpu
from jax.experimental.pallas import tpu_sc as plsc

N_NODES = 10000
N_EDGES = 320000
D_FEAT = 128
HIDDEN = 128
N_CLASSES = 40
C_PAD = 48

NC = 2
NS = 16
NW = NC * NS
N_PAD = 10240
ROWS_PER_SUB = N_PAD // NS
EPW = N_EDGES // NW
NBUF = 5
PREF = 3
EB_WIDE = 40
EB_NARROW = 80
DEG_W = 16

_MESH = plsc.VectorSubcoreMesh(core_axis_name="c", subcore_axis_name="s")
_SC_PARAMS = pltpu.CompilerParams(use_tc_tiling_on_sc=False)
_SC_PARAMS_NOLAYOUT = pltpu.CompilerParams(
    use_tc_tiling_on_sc=False, needs_layout_passes=False)


def _ew_slot(eb):
    return eb if eb % 16 == 0 else (eb // 16) * 16 + 16


def _zero_acc(zbuf, acc, s, eb, width):
    def zrow(e, carry):
        for j in range(width // 16):
            zbuf[e, pl.ds(j * 16, 16)] = jnp.zeros((16,), jnp.float32)
        return carry

    lax.fori_loop(0, eb, zrow, 0)

    def zcopy(i, carry):
        pltpu.sync_copy(zbuf, acc.at[pl.ds(s * ROWS_PER_SUB + i * eb, eb), :])
        return carry

    lax.fori_loop(0, ROWS_PER_SUB // eb, zcopy, 0)


def _scale_rows(rows, ewb, b, eb, width):
    def blk_body(blk, carry):
        wv = ewb[b, pl.ds(blk * 16, 16)]
        for k in range(16):
            e = blk * 16 + k
            w = wv[k]
            for j in range(width // 16):
                rows[b, e, pl.ds(j * 16, 16)] = (
                    rows[b, e, pl.ds(j * 16, 16)] * w)
        return carry

    lax.fori_loop(0, eb // 16, blk_body, 0)
    if eb % 16:
        wv = ewb[b, pl.ds((eb // 16) * 16, 16)]
        for k in range(eb % 16):
            e = (eb // 16) * 16 + k
            w = wv[k]
            for j in range(width // 16):
                rows[b, e, pl.ds(j * 16, 16)] = (
                    rows[b, e, pl.ds(j * 16, 16)] * w)


def _fill_rows(rows, ewb, b, eb):
    def blk_body(blk, carry):
        wv = ewb[b, pl.ds(blk * 16, 16)]
        for k in range(16):
            rows[b, blk * 16 + k, :] = jnp.full((DEG_W,), wv[k], jnp.float32)
        return carry

    lax.fori_loop(0, eb // 16, blk_body, 0)
    if eb % 16:
        wv = ewb[b, pl.ds((eb // 16) * 16, 16)]
        for k in range(eb % 16):
            rows[b, (eb // 16) * 16 + k, :] = jnp.full(
                (DEG_W,), wv[k], jnp.float32)


def _export(acc, out_hbm, c, s):
    pltpu.sync_copy(
        acc.at[pl.ds(s * ROWS_PER_SUB, ROWS_PER_SUB), :],
        out_hbm.at[c, pl.ds(s * ROWS_PER_SUB, ROWS_PER_SUB), :],
    )


def _make_msg_kernel(width, eb):
    nchunk = EPW // eb
    ews = _ew_slot(eb)

    @functools.partial(
        pl.kernel,
        out_type=jax.ShapeDtypeStruct((NC, N_PAD, width), jnp.float32),
        mesh=_MESH,
        scratch_types=[
            pltpu.VMEM((nchunk, eb), jnp.int32),
            pltpu.VMEM((nchunk, eb), jnp.int32),
            pltpu.VMEM((NBUF, ews), jnp.float32),
            pltpu.VMEM((NBUF, eb, width), jnp.float32),
            pltpu.VMEM_SHARED((N_PAD, width), jnp.float32),
            pltpu.SemaphoreType.DMA((NBUF,)),
            pltpu.SemaphoreType.DMA((NBUF,)),
        ],
        compiler_params=_SC_PARAMS,
    )
    def msg(g_hbm, src_hbm, dst_hbm, ew_hbm, out_hbm,
            src2, dst2, ewb, rows, acc, glsem, ssem):
        c = lax.axis_index("c")
        s = lax.axis_index("s")
        wid = s * NC + c
        ebase = wid * EPW

        _zero_acc(rows.at[0], acc, s, eb, width)
        plsc.subcore_barrier()

        pltpu.sync_copy(src_hbm.at[wid], src2)
        pltpu.sync_copy(dst_hbm.at[wid], dst2)

        def fire(j, bj):
            pltpu.async_copy(ew_hbm.at[pl.ds(ebase + j * eb, eb)],
                             ewb.at[bj, pl.ds(0, eb)], glsem.at[bj])
            pltpu.async_copy(g_hbm.at[src2.at[j]], rows.at[bj],
                             glsem.at[bj])

        def wait_gl(b):
            pltpu.make_async_copy(ew_hbm.at[pl.ds(0, eb)],
                                  ewb.at[b, pl.ds(0, eb)],
                                  glsem.at[b]).wait()
            pltpu.make_async_copy(g_hbm.at[pl.ds(0, eb), :], rows.at[b],
                                  glsem.at[b]).wait()

        def wait_s(b):
            pltpu.make_async_copy(g_hbm.at[pl.ds(0, eb), :], rows.at[b],
                                  ssem.at[b]).wait()

        for j in range(PREF):
            fire(j, j)

        def outer(u, carry):
            for b in range(NBUF):
                i = u * NBUF + b
                j = i + PREF
                bj = (b + PREF) % NBUF

                @pl.when(j < nchunk)
                def _():
                    @pl.when(j >= NBUF)
                    def _():
                        wait_s(bj)

                    fire(j, bj)

                wait_gl(b)
                _scale_rows(rows, ewb, b, eb, width)
                pltpu.async_copy(rows.at[b], acc.at[dst2.at[i]],
                                 ssem.at[b], add=True)
            return carry

        lax.fori_loop(0, nchunk // NBUF, outer, 0)
        for b in range(NBUF):
            wait_s(b)

        plsc.subcore_barrier()
        _export(acc, out_hbm, c, s)

    return msg


def _make_deg_kernel(eb):
    nchunk = EPW // eb
    ews = _ew_slot(eb)

    @functools.partial(
        pl.kernel,
        out_type=jax.ShapeDtypeStruct((NC, N_PAD, DEG_W), jnp.float32),
        mesh=_MESH,
        scratch_types=[
            pltpu.VMEM((nchunk, eb), jnp.int32),
            pltpu.VMEM((NBUF, ews), jnp.float32),
            pltpu.VMEM((NBUF, eb, DEG_W), jnp.float32),
            pltpu.VMEM_SHARED((N_PAD, DEG_W), jnp.float32),
            pltpu.SemaphoreType.DMA((NBUF,)),
            pltpu.SemaphoreType.DMA((NBUF,)),
        ],
        compiler_params=_SC_PARAMS,
    )
    def deg(ew_hbm, dst_hbm, out_hbm, dst2, ewb, rows, acc, lsem, ssem):
        c = lax.axis_index("c")
        s = lax.axis_index("s")
        wid = s * NC + c
        ebase = wid * EPW

        _zero_acc(rows.at[0], acc, s, eb, DEG_W)
        plsc.subcore_barrier()

        pltpu.sync_copy(dst_hbm.at[wid], dst2)

        def fire(j, bj):
            pltpu.async_copy(ew_hbm.at[pl.ds(ebase + j * eb, eb)],
                             ewb.at[bj, pl.ds(0, eb)], lsem.at[bj])

        def wait_l(b):
            pltpu.make_async_copy(ew_hbm.at[pl.ds(0, eb)],
                                  ewb.at[b, pl.ds(0, eb)],
                                  lsem.at[b]).wait()

        def wait_s(b):
            pltpu.make_async_copy(out_hbm.at[0, pl.ds(0, eb), :],
                                  rows.at[b], ssem.at[b]).wait()

        for j in range(PREF):
            fire(j, j)

        def outer(u, carry):
            for b in range(NBUF):
                i = u * NBUF + b
                j = i + PREF
                bj = (b + PREF) % NBUF

                @pl.when(j < nchunk)
                def _():
                    fire(j, bj)

                wait_l(b)

                @pl.when(i >= NBUF)
                def _():
                    wait_s(b)

                _fill_rows(rows, ewb, b, eb)
                pltpu.async_copy(rows.at[b], acc.at[dst2.at[i]],
                                 ssem.at[b], add=True)
            return carry

        lax.fori_loop(0, nchunk // NBUF, outer, 0)
        for b in range(NBUF):
            wait_s(b)

        plsc.subcore_barrier()
        _export(acc, out_hbm, c, s)

    return deg


_deg_kernel = _make_deg_kernel(EB_NARROW)
_msg48 = _make_msg_kernel(C_PAD, EB_NARROW)


_NBLK = 5
_BSUP = (EPW // EB_WIDE) // _NBLK
_GR = 5
_SR = 5


@functools.partial(
    pl.kernel,
    out_type=jax.ShapeDtypeStruct((NC, N_PAD, HIDDEN), jnp.float32),
    mesh=_MESH,
    scratch_types=[
        pltpu.VMEM((2, _BSUP, EB_WIDE), jnp.int32),
        pltpu.VMEM((2, _BSUP, EB_WIDE), jnp.int32),
        pltpu.VMEM((_GR, _ew_slot(EB_WIDE)), jnp.float32),
        pltpu.VMEM((_GR, EB_WIDE, HIDDEN), jnp.bfloat16),
        pltpu.VMEM((_SR, EB_WIDE, HIDDEN), jnp.float32),
        pltpu.VMEM_SHARED((N_PAD, HIDDEN), jnp.float32),
        pltpu.SemaphoreType.DMA((_GR,)),
        pltpu.SemaphoreType.DMA((_SR,)),
        pltpu.SemaphoreType.DMA((2,)),
    ],
    compiler_params=_SC_PARAMS_NOLAYOUT,
)
def _msg128(g_hbm, src_hbm, dst_hbm, ew_hbm, out_hbm,
            srcb, dstb, ewb, brows, stage, acc, glsem, ssem, isem):
    eb = EB_WIDE
    c = lax.axis_index("c")
    s = lax.axis_index("s")
    wid = s * NC + c
    ebase = wid * EPW

    _zero_acc(stage.at[0], acc, s, eb, HIDDEN)
    plsc.subcore_barrier()

    pltpu.sync_copy(src_hbm.at[wid, 0], srcb.at[0])
    pltpu.sync_copy(dst_hbm.at[wid, 0], dstb.at[0])

    def fire(j, row_ref, slot):
        pltpu.async_copy(ew_hbm.at[pl.ds(ebase + j * eb, eb)],
                         ewb.at[slot, pl.ds(0, eb)], glsem.at[slot])
        pltpu.async_copy(g_hbm.at[row_ref], brows.at[slot],
                         glsem.at[slot])

    def wait_gl(b):
        pltpu.make_async_copy(ew_hbm.at[pl.ds(0, eb)],
                              ewb.at[b, pl.ds(0, eb)],
                              glsem.at[b]).wait()
        pltpu.make_async_copy(g_hbm.at[pl.ds(0, eb), :], brows.at[b],
                              glsem.at[b]).wait()

    def wait_s(b):
        pltpu.make_async_copy(out_hbm.at[0, pl.ds(0, eb), :],
                              stage.at[b], ssem.at[b]).wait()

    def convert_scale(b):
        def sub_body(sb, carry):
            wv = ewb[b, pl.ds(sb * 8, 16)]
            for k in range(8):
                e = sb * 8 + k
                w = wv[k]
                for grp in range(HIDDEN // 32):
                    v = brows[b, e, pl.ds(grp * 32, 32)]
                    lo, hi = plsc.unpack(
                        v, format=plsc.PackFormat.INTERLEAVED)
                    stage[b, e, pl.ds(grp * 32, 16)] = lo * w
                    stage[b, e, pl.ds(grp * 32 + 16, 16)] = hi * w
            return carry

        lax.fori_loop(0, eb // 8, sub_body, 0)

    for j in range(PREF):
        fire(j, srcb.at[0, j], j)

    def blk_body(blk, carry):
        even = blk % 2 == 0

        def per_buf(buf_fn):
            @pl.when(even)
            def _():
                buf_fn(0, 1)

            @pl.when(jnp.logical_not(even))
            def _():
                buf_fn(1, 0)

        def u_body(u, carry2):
            for b in range(5):
                i = blk * _BSUP + 5 * u + b
                t = 5 * u + b

                if b == 0:
                    @pl.when(jnp.logical_and(u == 1, blk < _NBLK - 1))
                    def _():
                        def pf(cur, nxt):
                            pltpu.async_copy(src_hbm.at[wid, blk + 1],
                                             srcb.at[nxt], isem.at[nxt])
                            pltpu.async_copy(dst_hbm.at[wid, blk + 1],
                                             dstb.at[nxt], isem.at[nxt])
                        per_buf(pf)

                    @pl.when(jnp.logical_and(u == 8, blk < _NBLK - 1))
                    def _():
                        def pw(cur, nxt):
                            pltpu.make_async_copy(src_hbm.at[wid, 0],
                                                  srcb.at[nxt],
                                                  isem.at[nxt]).wait()
                            pltpu.make_async_copy(dst_hbm.at[wid, 0],
                                                  dstb.at[nxt],
                                                  isem.at[nxt]).wait()
                        per_buf(pw)

                if b < 2:
                    def fsame(cur, nxt, t=t, i=i, b=b):
                        fire(i + 3, srcb.at[cur, t + 3], (b + 3) % _GR)
                    per_buf(fsame)
                else:
                    @pl.when(u < 9)
                    def _(b=b, t=t, i=i):
                        def fsame(cur, nxt):
                            fire(i + 3, srcb.at[cur, t + 3], (b + 3) % _GR)
                        per_buf(fsame)

                    @pl.when(jnp.logical_and(u == 9, blk < _NBLK - 1))
                    def _(b=b, i=i):
                        def fnext(cur, nxt):
                            fire(i + 3, srcb.at[nxt, b - 2], (b + 3) % _GR)
                        per_buf(fnext)

                wait_gl(b)

                @pl.when(blk * 10 + u >= 1)
                def _(b=b):
                    wait_s(b)

                convert_scale(b)

                def fscat(cur, nxt, t=t, b=b):
                    pltpu.async_copy(stage.at[b], acc.at[dstb.at[cur, t]],
                                     ssem.at[b], add=True)
                per_buf(fscat)
            return carry2

        lax.fori_loop(0, _BSUP // 5, u_body, 0)
        return carry

    lax.fori_loop(0, _NBLK, blk_body, 0)

    for b in range(_SR):
        wait_s(b)

    plsc.subcore_barrier()
    _export(acc, out_hbm, c, s)


_BN = 2000


def _mm1_body(x_ref, w_ref, wp_ref, d0_ref, d1_ref, dis_ref, g_ref, gbf_ref):
    x = x_ref[...]
    h = jnp.dot(x, w_ref[...], preferred_element_type=jnp.float32)
    hp = jnp.dot(x, wp_ref[...], preferred_element_type=jnp.float32)
    dis = lax.rsqrt(d0_ref[...] + d1_ref[...] + 1.0)
    dis_ref[...] = dis
    g_ref[...] = dis * h
    gbf_ref[...] = (dis * hp).astype(jnp.bfloat16)


_PERM = []
for _j in range(HIDDEN // 32):
    for _k in range(16):
        _PERM.append(32 * _j + _k)
        _PERM.append(32 * _j + 16 + _k)


def _mid_body(a0_ref, a1_ref, g1_ref, dis_ref, b1_ref, w2_ref, g2_ref):
    dis = dis_ref[...]
    h = dis * (a0_ref[...] + a1_ref[...] + g1_ref[...]) + b1_ref[...]
    h = jnp.maximum(h, 0.0)
    h2 = jnp.dot(h, w2_ref[...], preferred_element_type=jnp.float32)
    g2_ref[...] = dis * h2


def _final_body(a0_ref, a1_ref, g2_ref, dis_ref, b2_ref, o_ref):
    o48 = dis_ref[...] * (a0_ref[...] + a1_ref[...] + g2_ref[...]) + b2_ref[...]
    o = o48[:, :N_CLASSES]
    m = jnp.max(o, axis=1, keepdims=True)
    lse = m + jnp.log(jnp.sum(jnp.exp(o - m), axis=1, keepdims=True))
    o_ref[...] = o - lse


def _rows_spec(width):
    return pl.BlockSpec((_BN, width), lambda i: (i, 0))


def _full_spec(shape):
    return pl.BlockSpec(shape, lambda i: tuple(0 for _ in shape))


def kernel(x, edge_index, edge_weight, W1, b1, W2, b2):
    src_w = edge_index[0].astype(jnp.int32).reshape(
        NW, _NBLK, _BSUP, EB_WIDE)
    dst_w = edge_index[1].astype(jnp.int32).reshape(
        NW, _NBLK, _BSUP, EB_WIDE)
    src_n = edge_index[0].astype(jnp.int32).reshape(
        NW, EPW // EB_NARROW, EB_NARROW)
    dst_n = edge_index[1].astype(jnp.int32).reshape(
        NW, EPW // EB_NARROW, EB_NARROW)
    ew = edge_weight.astype(jnp.float32)

    deg_parts = _deg_kernel(ew, dst_n)
    d0 = deg_parts[0, :N_NODES, 0:1]
    d1 = deg_parts[1, :N_NODES, 0:1]

    W1p = W1[:, jnp.array(_PERM, dtype=jnp.int32)]
    grid = N_NODES // _BN
    dis, g1, g1bf = pl.pallas_call(
        _mm1_body,
        grid=(grid,),
        in_specs=[
            _rows_spec(D_FEAT), _full_spec((D_FEAT, HIDDEN)),
            _full_spec((D_FEAT, HIDDEN)),
            _rows_spec(1), _rows_spec(1),
        ],
        out_specs=[_rows_spec(1), _rows_spec(HIDDEN), _rows_spec(HIDDEN)],
        out_shape=[
            jax.ShapeDtypeStruct((N_NODES, 1), jnp.float32),
            jax.ShapeDtypeStruct((N_NODES, HIDDEN), jnp.float32),
            jax.ShapeDtypeStruct((N_NODES, HIDDEN), jnp.bfloat16),
        ],
    )(x, W1, W1p, d0, d1)

    a1 = _msg128(g1bf, src_w, dst_w, ew)

    W2p = jnp.pad(W2, ((0, 0), (0, C_PAD - N_CLASSES)))
    b1r = b1.reshape(1, HIDDEN)
    b2r = jnp.pad(b2, (0, C_PAD - N_CLASSES)).reshape(1, C_PAD)

    g2 = pl.pallas_call(
        _mid_body,
        grid=(grid,),
        in_specs=[
            _rows_spec(HIDDEN), _rows_spec(HIDDEN), _rows_spec(HIDDEN),
            _rows_spec(1), _full_spec((1, HIDDEN)),
            _full_spec((HIDDEN, C_PAD)),
        ],
        out_specs=_rows_spec(C_PAD),
        out_shape=jax.ShapeDtypeStruct((N_NODES, C_PAD), jnp.float32),
    )(a1[0, :N_NODES], a1[1, :N_NODES], g1, dis, b1r, W2p)

    a2 = _msg48(g2, src_n, dst_n, ew)

    out = pl.pallas_call(
        _final_body,
        grid=(grid,),
        in_specs=[
            _rows_spec(C_PAD), _rows_spec(C_PAD), _rows_spec(C_PAD),
            _rows_spec(1), _full_spec((1, C_PAD)),
        ],
        out_specs=_rows_spec(N_CLASSES),
        out_shape=jax.ShapeDtypeStruct((N_NODES, N_CLASSES), jnp.float32),
    )(a2[0, :N_NODES], a2[1, :N_NODES], g2, dis, b2r)

    return out


# speedup vs baseline: 1.6172x; 1.6172x over previous
import functools

import jax
import jax.numpy as jnp
from jax import lax
from jax.experimental import pallas as pl
from jax.experimental.pallas import tpu as pltpu
from jax.experimental.pallas import tpu_sc as plsc

N_NODES = 10000
N_EDGES = 320000
D_FEAT = 128
HIDDEN = 128
N_CLASSES = 40
C_PAD = 48

NC = 2
NS = 16
NW = NC * NS
N_PAD = 10240
ROWS_PER_SUB = N_PAD // NS
EPW = N_EDGES // NW
NBUF = 5
PREF = 3
EB_WIDE = 40
EB_NARROW = 80
DEG_W = 16

_MESH = plsc.VectorSubcoreMesh(core_axis_name="c", subcore_axis_name="s")
_SC_PARAMS = pltpu.CompilerParams(use_tc_tiling_on_sc=False)


def _ew_slot(eb):
    return eb if eb % 16 == 0 else (eb // 16) * 16 + 16


def _zero_acc(zbuf, acc, s, eb, width):
    def zrow(e, carry):
        for j in range(width // 16):
            zbuf[e, pl.ds(j * 16, 16)] = jnp.zeros((16,), jnp.float32)
        return carry

    lax.fori_loop(0, eb, zrow, 0)

    def zcopy(i, carry):
        pltpu.sync_copy(zbuf, acc.at[pl.ds(s * ROWS_PER_SUB + i * eb, eb), :])
        return carry

    lax.fori_loop(0, ROWS_PER_SUB // eb, zcopy, 0)


def _scale_rows(rows, ewb, b, eb, width):
    def blk_body(blk, carry):
        wv = ewb[b, pl.ds(blk * 16, 16)]
        for k in range(16):
            e = blk * 16 + k
            w = wv[k]
            for j in range(width // 16):
                rows[b, e, pl.ds(j * 16, 16)] = (
                    rows[b, e, pl.ds(j * 16, 16)] * w)
        return carry

    lax.fori_loop(0, eb // 16, blk_body, 0)
    if eb % 16:
        wv = ewb[b, pl.ds((eb // 16) * 16, 16)]
        for k in range(eb % 16):
            e = (eb // 16) * 16 + k
            w = wv[k]
            for j in range(width // 16):
                rows[b, e, pl.ds(j * 16, 16)] = (
                    rows[b, e, pl.ds(j * 16, 16)] * w)


def _fill_rows(rows, ewb, b, eb):
    def blk_body(blk, carry):
        wv = ewb[b, pl.ds(blk * 16, 16)]
        for k in range(16):
            rows[b, blk * 16 + k, :] = jnp.full((DEG_W,), wv[k], jnp.float32)
        return carry

    lax.fori_loop(0, eb // 16, blk_body, 0)
    if eb % 16:
        wv = ewb[b, pl.ds((eb // 16) * 16, 16)]
        for k in range(eb % 16):
            rows[b, (eb // 16) * 16 + k, :] = jnp.full(
                (DEG_W,), wv[k], jnp.float32)


def _export(acc, out_hbm, c, s):
    pltpu.sync_copy(
        acc.at[pl.ds(s * ROWS_PER_SUB, ROWS_PER_SUB), :],
        out_hbm.at[c, pl.ds(s * ROWS_PER_SUB, ROWS_PER_SUB), :],
    )


def _make_msg_kernel(width, eb):
    nchunk = EPW // eb
    ews = _ew_slot(eb)

    @functools.partial(
        pl.kernel,
        out_type=jax.ShapeDtypeStruct((NC, N_PAD, width), jnp.float32),
        mesh=_MESH,
        scratch_types=[
            pltpu.VMEM((nchunk, eb), jnp.int32),
            pltpu.VMEM((nchunk, eb), jnp.int32),
            pltpu.VMEM((NBUF, ews), jnp.float32),
            pltpu.VMEM((NBUF, eb, width), jnp.float32),
            pltpu.VMEM_SHARED((N_PAD, width), jnp.float32),
            pltpu.SemaphoreType.DMA((NBUF,)),
            pltpu.SemaphoreType.DMA((NBUF,)),
        ],
        compiler_params=_SC_PARAMS,
    )
    def msg(g_hbm, src_hbm, dst_hbm, ew_hbm, out_hbm,
            src2, dst2, ewb, rows, acc, glsem, ssem):
        c = lax.axis_index("c")
        s = lax.axis_index("s")
        wid = s * NC + c
        ebase = wid * EPW

        _zero_acc(rows.at[0], acc, s, eb, width)
        plsc.subcore_barrier()

        pltpu.sync_copy(src_hbm.at[wid], src2)
        pltpu.sync_copy(dst_hbm.at[wid], dst2)

        def fire(j, bj):
            pltpu.async_copy(ew_hbm.at[pl.ds(ebase + j * eb, eb)],
                             ewb.at[bj, pl.ds(0, eb)], glsem.at[bj])
            pltpu.async_copy(g_hbm.at[src2.at[j]], rows.at[bj],
                             glsem.at[bj])

        def wait_gl(b):
            pltpu.make_async_copy(ew_hbm.at[pl.ds(0, eb)],
                                  ewb.at[b, pl.ds(0, eb)],
                                  glsem.at[b]).wait()
            pltpu.make_async_copy(g_hbm.at[pl.ds(0, eb), :], rows.at[b],
                                  glsem.at[b]).wait()

        def wait_s(b):
            pltpu.make_async_copy(g_hbm.at[pl.ds(0, eb), :], rows.at[b],
                                  ssem.at[b]).wait()

        for j in range(PREF):
            fire(j, j)

        def outer(u, carry):
            for b in range(NBUF):
                i = u * NBUF + b
                j = i + PREF
                bj = (b + PREF) % NBUF

                @pl.when(j < nchunk)
                def _():
                    @pl.when(j >= NBUF)
                    def _():
                        wait_s(bj)

                    fire(j, bj)

                wait_gl(b)
                _scale_rows(rows, ewb, b, eb, width)
                pltpu.async_copy(rows.at[b], acc.at[dst2.at[i]],
                                 ssem.at[b], add=True)
            return carry

        lax.fori_loop(0, nchunk // NBUF, outer, 0)
        for b in range(NBUF):
            wait_s(b)

        plsc.subcore_barrier()
        _export(acc, out_hbm, c, s)

    return msg


def _make_deg_kernel(eb):
    nchunk = EPW // eb
    ews = _ew_slot(eb)

    @functools.partial(
        pl.kernel,
        out_type=jax.ShapeDtypeStruct((NC, N_PAD, DEG_W), jnp.float32),
        mesh=_MESH,
        scratch_types=[
            pltpu.VMEM((nchunk, eb), jnp.int32),
            pltpu.VMEM((NBUF, ews), jnp.float32),
            pltpu.VMEM((NBUF, eb, DEG_W), jnp.float32),
            pltpu.VMEM_SHARED((N_PAD, DEG_W), jnp.float32),
            pltpu.SemaphoreType.DMA((NBUF,)),
            pltpu.SemaphoreType.DMA((NBUF,)),
        ],
        compiler_params=_SC_PARAMS,
    )
    def deg(ew_hbm, dst_hbm, out_hbm, dst2, ewb, rows, acc, lsem, ssem):
        c = lax.axis_index("c")
        s = lax.axis_index("s")
        wid = s * NC + c
        ebase = wid * EPW

        _zero_acc(rows.at[0], acc, s, eb, DEG_W)
        plsc.subcore_barrier()

        pltpu.sync_copy(dst_hbm.at[wid], dst2)

        def fire(j, bj):
            pltpu.async_copy(ew_hbm.at[pl.ds(ebase + j * eb, eb)],
                             ewb.at[bj, pl.ds(0, eb)], lsem.at[bj])

        def wait_l(b):
            pltpu.make_async_copy(ew_hbm.at[pl.ds(0, eb)],
                                  ewb.at[b, pl.ds(0, eb)],
                                  lsem.at[b]).wait()

        def wait_s(b):
            pltpu.make_async_copy(out_hbm.at[0, pl.ds(0, eb), :],
                                  rows.at[b], ssem.at[b]).wait()

        for j in range(PREF):
            fire(j, j)

        def outer(u, carry):
            for b in range(NBUF):
                i = u * NBUF + b
                j = i + PREF
                bj = (b + PREF) % NBUF

                @pl.when(j < nchunk)
                def _():
                    fire(j, bj)

                wait_l(b)

                @pl.when(i >= NBUF)
                def _():
                    wait_s(b)

                _fill_rows(rows, ewb, b, eb)
                pltpu.async_copy(rows.at[b], acc.at[dst2.at[i]],
                                 ssem.at[b], add=True)
            return carry

        lax.fori_loop(0, nchunk // NBUF, outer, 0)
        for b in range(NBUF):
            wait_s(b)

        plsc.subcore_barrier()
        _export(acc, out_hbm, c, s)

    return deg


_deg_kernel = _make_deg_kernel(EB_NARROW)
_msg128 = _make_msg_kernel(HIDDEN, EB_WIDE)
_msg48 = _make_msg_kernel(C_PAD, EB_NARROW)


_BN = 2000


def _mm1_body(x_ref, w_ref, dp_ref, dis_ref, g_ref):
    h = jnp.dot(x_ref[...], w_ref[...], preferred_element_type=jnp.float32)
    deg = dp_ref[0, :, 0:1] + dp_ref[1, :, 0:1] + 1.0
    dis = lax.rsqrt(deg)
    dis_ref[...] = dis
    g_ref[...] = dis * h


def _mid_body(a_ref, g1_ref, dis_ref, b1_ref, w2_ref, g2_ref):
    dis = dis_ref[...]
    h = dis * (a_ref[0] + a_ref[1] + g1_ref[...]) + b1_ref[...]
    h = jnp.maximum(h, 0.0)
    h2 = jnp.dot(h, w2_ref[...], preferred_element_type=jnp.float32)
    g2_ref[...] = dis * h2


def _final_body(a_ref, g2_ref, dis_ref, b2_ref, o_ref):
    o48 = dis_ref[...] * (a_ref[0] + a_ref[1] + g2_ref[...]) + b2_ref[...]
    o = o48[:, :N_CLASSES]
    m = jnp.max(o, axis=1, keepdims=True)
    lse = m + jnp.log(jnp.sum(jnp.exp(o - m), axis=1, keepdims=True))
    o_ref[...] = o - lse


def _rows_spec(width):
    return pl.BlockSpec((_BN, width), lambda i: (i, 0))


def _full_spec(shape):
    return pl.BlockSpec(shape, lambda i: tuple(0 for _ in shape))


def kernel(x, edge_index, edge_weight, W1, b1, W2, b2):
    src_w = edge_index[0].astype(jnp.int32).reshape(NW, EPW // EB_WIDE, EB_WIDE)
    dst_w = edge_index[1].astype(jnp.int32).reshape(NW, EPW // EB_WIDE, EB_WIDE)
    src_n = edge_index[0].astype(jnp.int32).reshape(
        NW, EPW // EB_NARROW, EB_NARROW)
    dst_n = edge_index[1].astype(jnp.int32).reshape(
        NW, EPW // EB_NARROW, EB_NARROW)
    ew = edge_weight.astype(jnp.float32)

    deg_parts = _deg_kernel(ew, dst_n)

    grid = N_NODES // _BN
    dis, g1 = pl.pallas_call(
        _mm1_body,
        grid=(grid,),
        in_specs=[
            _rows_spec(D_FEAT), _full_spec((D_FEAT, HIDDEN)),
            pl.BlockSpec((NC, _BN, DEG_W), lambda i: (0, i, 0)),
        ],
        out_specs=[_rows_spec(1), _rows_spec(HIDDEN)],
        out_shape=[
            jax.ShapeDtypeStruct((N_NODES, 1), jnp.float32),
            jax.ShapeDtypeStruct((N_NODES, HIDDEN), jnp.float32),
        ],
    )(x, W1, deg_parts)

    a1 = _msg128(g1, src_w, dst_w, ew)

    W2p = jnp.pad(W2, ((0, 0), (0, C_PAD - N_CLASSES)))
    b1r = b1.reshape(1, HIDDEN)
    b2r = jnp.pad(b2, (0, C_PAD - N_CLASSES)).reshape(1, C_PAD)

    g2 = pl.pallas_call(
        _mid_body,
        grid=(grid,),
        in_specs=[
            pl.BlockSpec((NC, _BN, HIDDEN), lambda i: (0, i, 0)),
            _rows_spec(HIDDEN),
            _rows_spec(1), _full_spec((1, HIDDEN)),
            _full_spec((HIDDEN, C_PAD)),
        ],
        out_specs=_rows_spec(C_PAD),
        out_shape=jax.ShapeDtypeStruct((N_NODES, C_PAD), jnp.float32),
    )(a1, g1, dis, b1r, W2p)

    a2 = _msg48(g2, src_n, dst_n, ew)

    out = pl.pallas_call(
        _final_body,
        grid=(grid,),
        in_specs=[
            pl.BlockSpec((NC, _BN, C_PAD), lambda i: (0, i, 0)),
            _rows_spec(C_PAD),
            _rows_spec(1), _full_spec((1, C_PAD)),
        ],
        out_specs=_rows_spec(N_CLASSES),
        out_shape=jax.ShapeDtypeStruct((N_NODES, N_CLASSES), jnp.float32),
    )(a2, g2, dis, b2r)

    return out
